# Initial kernel scaffold; baseline (speedup 1.0000x reference)
import jax
import jax.numpy as jnp
from jax.experimental import pallas as pl


def kernel(x, edge_index, edge_attr, lin0_w, lin0_b, lin_h_w, lin_h_b, lin_hm_w, lin_hm_b, en1_w, en1_b, en2_w, en2_b, conv_b):
    raise NotImplementedError("write your pallas kernel here")


# speedup vs baseline: 1.0142x; 1.0142x over previous
import functools

import jax
import jax.numpy as jnp
from jax.experimental import pallas as pl
from jax.experimental.pallas import tpu as pltpu

N = 10000
E = 160000
HID = 128
D = 64

EB = 1280
NB = 1000


def _node0_body(x_ref, w_ref, b_ref, o_ref):
    o_ref[...] = jax.nn.relu(
        jnp.dot(x_ref[...], w_ref[...], preferred_element_type=jnp.float32)
        + b_ref[...]
    )


def _hid_body(ea_ref, w_ref, b_ref, o_ref):
    o_ref[...] = jax.nn.relu(
        jnp.dot(ea_ref[...], w_ref[...], preferred_element_type=jnp.float32)
        + b_ref[...]
    )


def _msg_body(hid_ref, xj_ref, w2_ref, bmat_ref, o_ref):
    hb = hid_ref[...].astype(jnp.bfloat16)
    xb = xj_ref[...].astype(jnp.bfloat16)
    z = (hb[:, :, None] * xb[:, None, :]).reshape(hb.shape[0], D * D)
    o_ref[...] = (
        jnp.dot(z, w2_ref[...], preferred_element_type=jnp.float32)
        + jnp.dot(xj_ref[...], bmat_ref[...], preferred_element_type=jnp.float32)
    )


def _update_body(agg_ref, cnt_ref, h_ref, whh_ref, bh_ref, whm_ref, bhm_ref,
                 cb_ref, out_ref, hn_ref, outr_ref):
    agg = agg_ref[0] + agg_ref[1]
    cnt = cnt_ref[0, :, :1] + cnt_ref[1, :, :1]
    denom = jnp.maximum(cnt, 1.0)
    m = jax.nn.relu(agg / denom + cb_ref[...])
    hn = jax.nn.relu(
        jnp.dot(h_ref[...], whh_ref[...], preferred_element_type=jnp.float32)
        + bh_ref[...]
    )
    o = jax.nn.relu(
        jnp.dot(hn, whm_ref[0:D, :], preferred_element_type=jnp.float32)
        + jnp.dot(m, whm_ref[D:2 * D, :], preferred_element_type=jnp.float32)
        + bhm_ref[...]
    ) + h_ref[...]
    out_ref[...] = o
    hn_ref[...] = hn
    outr_ref[...] = jax.nn.relu(o)


def _node0(x, w, b):
    return pl.pallas_call(
        _node0_body,
        grid=(N // NB,),
        in_specs=[
            pl.BlockSpec((NB, HID), lambda i: (i, 0)),
            pl.BlockSpec((HID, D), lambda i: (0, 0)),
            pl.BlockSpec((1, D), lambda i: (0, 0)),
        ],
        out_specs=pl.BlockSpec((NB, D), lambda i: (i, 0)),
        out_shape=jax.ShapeDtypeStruct((N, D), jnp.float32),
    )(x, w, b)


def _hid(ea, w, b):
    ca = ea.shape[1]
    return pl.pallas_call(
        _hid_body,
        grid=(E // 2000,),
        in_specs=[
            pl.BlockSpec((2000, ca), lambda i: (i, 0)),
            pl.BlockSpec((ca, D), lambda i: (0, 0)),
            pl.BlockSpec((1, D), lambda i: (0, 0)),
        ],
        out_specs=pl.BlockSpec((2000, D), lambda i: (i, 0)),
        out_shape=jax.ShapeDtypeStruct((E, D), jnp.float32),
    )(ea, w, b)


def _msg(hid, xj, w2_bf, bmat):
    return pl.pallas_call(
        _msg_body,
        grid=(E // EB,),
        in_specs=[
            pl.BlockSpec((EB, D), lambda i: (i, 0)),
            pl.BlockSpec((EB, D), lambda i: (i, 0)),
            pl.BlockSpec((D * D, D), lambda i: (0, 0)),
            pl.BlockSpec((D, D), lambda i: (0, 0)),
        ],
        out_specs=pl.BlockSpec((EB, D), lambda i: (i, 0)),
        out_shape=jax.ShapeDtypeStruct((E, D), jnp.float32),
    )(hid, xj, w2_bf, bmat)


def _update(agg2, cnt2, h, whh, bh, whm, bhm, cb):
    return pl.pallas_call(
        _update_body,
        grid=(N // NB,),
        in_specs=[
            pl.BlockSpec((2, NB, D), lambda i: (0, i, 0)),
            pl.BlockSpec((2, NB, 16), lambda i: (0, i, 0)),
            pl.BlockSpec((NB, D), lambda i: (i, 0)),
            pl.BlockSpec((D, D), lambda i: (0, 0)),
            pl.BlockSpec((1, D), lambda i: (0, 0)),
            pl.BlockSpec((2 * D, D), lambda i: (0, 0)),
            pl.BlockSpec((1, D), lambda i: (0, 0)),
            pl.BlockSpec((1, D), lambda i: (0, 0)),
        ],
        out_specs=[
            pl.BlockSpec((NB, D), lambda i: (i, 0)),
            pl.BlockSpec((NB, D), lambda i: (i, 0)),
            pl.BlockSpec((NB, D), lambda i: (i, 0)),
        ],
        out_shape=[
            jax.ShapeDtypeStruct((N, D), jnp.float32),
            jax.ShapeDtypeStruct((N, D), jnp.float32),
            jax.ShapeDtypeStruct((N, D), jnp.float32),
        ],
    )(agg2, cnt2, h, whh, bh, whm, bhm, cb)


def kernel(x, edge_index, edge_attr, lin0_w, lin0_b, lin_h_w, lin_h_b,
           lin_hm_w, lin_hm_b, en1_w, en1_b, en2_w, en2_b, conv_b):
    src = edge_index[0]
    dst = edge_index[1]

    w2_bf = en2_w.reshape(D, D, D).reshape(D * D, D).astype(jnp.bfloat16)
    bmat = en2_b.reshape(D, D)

    hid = _hid(edge_attr, en1_w, en1_b[None, :])
    out = _node0(x, lin0_w, lin0_b[None, :])

    ones = jnp.ones((E, 16), jnp.float32)
    cnt = jax.ops.segment_sum(ones, dst, num_segments=N)
    cnt2 = jnp.stack([cnt, jnp.zeros_like(cnt)])

    h = out
    for _ in range(2):
        xj = out[src]
        msg = _msg(hid, xj, w2_bf, bmat)
        agg = jax.ops.segment_sum(msg, dst, num_segments=N)
        agg2 = jnp.stack([agg, jnp.zeros_like(agg)])
        out, h, outr = _update(agg2, cnt2, h, lin_h_w, lin_h_b[None, :],
                               lin_hm_w, lin_hm_b[None, :], conv_b[None, :])
    return outr


# speedup vs baseline: 2.4425x; 2.4082x over previous
import functools

import jax
import jax.numpy as jnp
from jax import lax
from jax.experimental import pallas as pl
from jax.experimental.pallas import tpu as pltpu
from jax.experimental.pallas import tpu_sc as plsc

N = 10000
E = 160000
HID = 128
D = 64

NSC = 2
NTEC = 16
LW = 128
EP = 163840
CPW = EP // (NSC * NTEC * LW)
NPAD = 10240
STRIPE = NPAD // NTEC

EB = 1024
NB = 1280
HB = 6400

_mesh = plsc.VectorSubcoreMesh(core_axis_name="c", subcore_axis_name="s")
_sc_params = pltpu.CompilerParams(use_tc_tiling_on_sc=False)


def _node0_body(x_ref, w_ref, b_ref, o_ref):
    o_ref[...] = jax.nn.relu(
        jnp.dot(x_ref[...], w_ref[...], preferred_element_type=jnp.float32)
        + b_ref[...]
    )


def _hid_body(ea_ref, w_ref, b_ref, o_ref):
    o_ref[...] = jax.nn.relu(
        jnp.dot(ea_ref[...], w_ref[...], preferred_element_type=jnp.float32)
        + b_ref[...]
    )


def _msg_body(hid_ref, xj_ref, et_ref, bmat_ref, o_ref):
    hbT = hid_ref[...].T.astype(jnp.bfloat16)
    wft = jnp.dot(et_ref[...], hbT, preferred_element_type=jnp.float32)
    xjT = xj_ref[...].T
    acc = xjT[0:1, :] * wft[0:D, :]
    for i in range(1, D):
        acc = acc + xjT[i:i + 1, :] * wft[i * D:(i + 1) * D, :]
    o_ref[...] = acc.T + jnp.dot(
        xj_ref[...], bmat_ref[...], preferred_element_type=jnp.float32)


def _update_body(agg_ref, cnt_ref, h_ref, whh_ref, bh_ref, whm_ref, bhm_ref,
                 cb_ref, out_ref, hn_ref, outr_ref):
    agg = agg_ref[0] + agg_ref[1]
    cnt = cnt_ref[0, :, :1] + cnt_ref[1, :, :1]
    denom = jnp.maximum(cnt, 1.0)
    m = jax.nn.relu(agg / denom + cb_ref[...])
    hn = jax.nn.relu(
        jnp.dot(h_ref[...], whh_ref[...], preferred_element_type=jnp.float32)
        + bh_ref[...]
    )
    o = jax.nn.relu(
        jnp.dot(hn, whm_ref[0:D, :], preferred_element_type=jnp.float32)
        + jnp.dot(m, whm_ref[D:2 * D, :], preferred_element_type=jnp.float32)
        + bhm_ref[...]
    ) + h_ref[...]
    out_ref[...] = o
    hn_ref[...] = hn
    outr_ref[...] = jax.nn.relu(o)


def _node0(x, w, b):
    return pl.pallas_call(
        _node0_body,
        grid=(N // 1000,),
        in_specs=[
            pl.BlockSpec((1000, HID), lambda i: (i, 0)),
            pl.BlockSpec((HID, D), lambda i: (0, 0)),
            pl.BlockSpec((1, D), lambda i: (0, 0)),
        ],
        out_specs=pl.BlockSpec((1000, D), lambda i: (i, 0)),
        out_shape=jax.ShapeDtypeStruct((NPAD, D), jnp.float32),
    )(x, w, b)


def _hid(ea, w, b):
    ca = ea.shape[1]
    return pl.pallas_call(
        _hid_body,
        grid=(E // HB,),
        in_specs=[
            pl.BlockSpec((HB, ca), lambda i: (i, 0)),
            pl.BlockSpec((ca, D), lambda i: (0, 0)),
            pl.BlockSpec((1, D), lambda i: (0, 0)),
        ],
        out_specs=pl.BlockSpec((HB, D), lambda i: (i, 0)),
        out_shape=jax.ShapeDtypeStruct((EP, D), jnp.float32),
    )(ea, w, b)


def _msg(hid, xj, et_bf, bmat):
    return pl.pallas_call(
        _msg_body,
        grid=(EP // EB,),
        in_specs=[
            pl.BlockSpec((EB, D), lambda i: (i, 0)),
            pl.BlockSpec((EB, D), lambda i: (i, 0)),
            pl.BlockSpec((D * D, D), lambda i: (0, 0)),
            pl.BlockSpec((D, D), lambda i: (0, 0)),
        ],
        out_specs=pl.BlockSpec((EB, D), lambda i: (i, 0)),
        out_shape=jax.ShapeDtypeStruct((EP, D), jnp.float32),
    )(hid, xj, et_bf, bmat)


def _update(agg2, cnt2, h, whh, bh, whm, bhm, cb):
    return pl.pallas_call(
        _update_body,
        grid=(NPAD // NB,),
        in_specs=[
            pl.BlockSpec((2, NB, D), lambda i: (0, i, 0)),
            pl.BlockSpec((2, NB, 16), lambda i: (0, i, 0)),
            pl.BlockSpec((NB, D), lambda i: (i, 0)),
            pl.BlockSpec((D, D), lambda i: (0, 0)),
            pl.BlockSpec((1, D), lambda i: (0, 0)),
            pl.BlockSpec((2 * D, D), lambda i: (0, 0)),
            pl.BlockSpec((1, D), lambda i: (0, 0)),
            pl.BlockSpec((1, D), lambda i: (0, 0)),
        ],
        out_specs=[
            pl.BlockSpec((NB, D), lambda i: (i, 0)),
            pl.BlockSpec((NB, D), lambda i: (i, 0)),
            pl.BlockSpec((NB, D), lambda i: (i, 0)),
        ],
        out_shape=[
            jax.ShapeDtypeStruct((NPAD, D), jnp.float32),
            jax.ShapeDtypeStruct((NPAD, D), jnp.float32),
            jax.ShapeDtypeStruct((NPAD, D), jnp.float32),
        ],
    )(agg2, cnt2, h, whh, bh, whm, bhm, cb)


@functools.partial(
    pl.kernel,
    mesh=_mesh,
    out_type=jax.ShapeDtypeStruct((EP, D), jnp.float32),
    scratch_types=[
        pltpu.VMEM((CPW * LW,), jnp.int32),
        pltpu.VMEM((2, LW, D), jnp.float32),
        pltpu.SemaphoreType.DMA,
        pltpu.SemaphoreType.DMA,
    ],
    compiler_params=_sc_params,
)
def _gather_sc(table_hbm, idx_hbm, out_hbm, idx_v, rows_v, sem0, sem1):
    w = lax.axis_index("c") * NTEC + lax.axis_index("s")
    base = w * CPW
    sems = (sem0, sem1)
    pltpu.sync_copy(idx_hbm.at[pl.ds(base * LW, CPW * LW)], idx_v)
    for b in range(2):
        pltpu.async_copy(
            table_hbm.at[idx_v.at[pl.ds(b * LW, LW)]], rows_v.at[b], sems[b])

    def body(i, _):
        def one(j, b):
            pltpu.make_async_copy(
                table_hbm.at[pl.ds(0, LW)], rows_v.at[b], sems[b]).wait()
            pltpu.sync_copy(rows_v.at[b], out_hbm.at[pl.ds((base + j) * LW, LW)])

            @pl.when(j + 2 < CPW)
            def _():
                pltpu.async_copy(
                    table_hbm.at[idx_v.at[pl.ds((j + 2) * LW, LW)]],
                    rows_v.at[b], sems[b])

        one(2 * i, 0)
        one(2 * i + 1, 1)
        return ()

    lax.fori_loop(0, CPW // 2, body, (), unroll=False)


@functools.partial(
    pl.kernel,
    mesh=_mesh,
    out_type=jax.ShapeDtypeStruct((NSC, NPAD, D), jnp.float32),
    scratch_types=[
        pltpu.VMEM((CPW, LW), jnp.int32),
        pltpu.VMEM((LW, D), jnp.float32),
        pltpu.VMEM_SHARED((NPAD, D), jnp.float32),
    ],
    compiler_params=_sc_params,
)
def _scatter_sc(msg_hbm, idx_hbm, zeros_hbm, out_hbm, idx_v, row_v, agg_sh):
    c = lax.axis_index("c")
    s = lax.axis_index("s")
    base = (c * NTEC + s) * CPW
    pltpu.sync_copy(zeros_hbm.at[pl.ds(s * STRIPE, STRIPE)],
                    agg_sh.at[pl.ds(s * STRIPE, STRIPE)])

    def load_idx(i, _):
        pltpu.sync_copy(idx_hbm.at[pl.ds((base + i) * LW, LW)], idx_v.at[i])
        return ()

    lax.fori_loop(0, CPW, load_idx, (), unroll=False)
    plsc.subcore_barrier()

    def body(i, _):
        pltpu.sync_copy(msg_hbm.at[pl.ds((base + i) * LW, LW)], row_v)
        pltpu.sync_copy(row_v, agg_sh.at[idx_v.at[i]], add=True)
        return ()

    lax.fori_loop(0, CPW, body, (), unroll=False)
    plsc.subcore_barrier()
    pltpu.sync_copy(agg_sh.at[pl.ds(s * STRIPE, STRIPE)],
                    out_hbm.at[c, pl.ds(s * STRIPE, STRIPE)])


@functools.partial(
    pl.kernel,
    mesh=_mesh,
    out_type=jax.ShapeDtypeStruct((NSC, NPAD, 16), jnp.float32),
    scratch_types=[
        pltpu.VMEM((CPW, LW), jnp.int32),
        pltpu.VMEM((LW, 16), jnp.float32),
        pltpu.VMEM_SHARED((NPAD, 16), jnp.float32),
    ],
    compiler_params=_sc_params,
)
def _count_sc(ones_hbm, idx_hbm, zeros_hbm, out_hbm, idx_v, ones_v, cnt_sh):
    c = lax.axis_index("c")
    s = lax.axis_index("s")
    base = (c * NTEC + s) * CPW
    pltpu.sync_copy(zeros_hbm.at[pl.ds(s * STRIPE, STRIPE)],
                    cnt_sh.at[pl.ds(s * STRIPE, STRIPE)])
    pltpu.sync_copy(ones_hbm, ones_v)

    def load_idx(i, _):
        pltpu.sync_copy(idx_hbm.at[pl.ds((base + i) * LW, LW)], idx_v.at[i])
        return ()

    lax.fori_loop(0, CPW, load_idx, (), unroll=False)
    plsc.subcore_barrier()

    def body(i, _):
        pltpu.sync_copy(ones_v, cnt_sh.at[idx_v.at[i]], add=True)
        return ()

    lax.fori_loop(0, CPW, body, (), unroll=False)
    plsc.subcore_barrier()
    pltpu.sync_copy(cnt_sh.at[pl.ds(s * STRIPE, STRIPE)],
                    out_hbm.at[c, pl.ds(s * STRIPE, STRIPE)])


def kernel(x, edge_index, edge_attr, lin0_w, lin0_b, lin_h_w, lin_h_b,
           lin_hm_w, lin_hm_b, en1_w, en1_b, en2_w, en2_b, conv_b):
    src = edge_index[0]
    dst = edge_index[1]

    src_p = jnp.pad(src, (0, EP - E))
    dst_p = jnp.pad(dst, (0, EP - E), constant_values=N)
    et_bf = en2_w.T.astype(jnp.bfloat16)
    bmat = en2_b.reshape(D, D)
    zeros_nd = jnp.zeros((NPAD, D), jnp.float32)
    zeros_n16 = jnp.zeros((NPAD, 16), jnp.float32)
    ones_rows = jnp.ones((LW, 16), jnp.float32)

    hid = _hid(edge_attr, en1_w, en1_b[None, :])
    out = _node0(x, lin0_w, lin0_b[None, :])
    cnt2 = _count_sc(ones_rows, dst_p, zeros_n16)

    h = out
    for _ in range(2):
        xj = _gather_sc(out, src_p)
        msg = _msg(hid, xj, et_bf, bmat)
        agg2 = _scatter_sc(msg, dst_p, zeros_nd)
        out, h, outr = _update(agg2, cnt2, h, lin_h_w, lin_h_b[None, :],
                               lin_hm_w, lin_hm_b[None, :], conv_b[None, :])
    return outr[:N]
